# Initial kernel scaffold; baseline (speedup 1.0000x reference)
import functools

import jax
import jax.numpy as jnp
from jax import lax
from jax.experimental import pallas as pl
from jax.experimental.pallas import tpu as pltpu
from jax.experimental.pallas import tpu_sc as plsc

B = 4096
L = 200
D = 64
DD = 3 * D
H = 256
C = 10

NC = 2
NS = 16
NW = NC * NS
ROWS_PER_W = B // NW
CB = 32
LANES = 16
G0 = 128
G1 = L - G0


def _pooled_sum_sc(x0, x1, x2, t0, t1, t2):
    mesh = plsc.VectorSubcoreMesh(core_axis_name="c", subcore_axis_name="s")

    @functools.partial(
        pl.kernel,
        out_type=jax.ShapeDtypeStruct((B, DD), jnp.float32),
        mesh=mesh,
        scratch_types=[
            pltpu.VMEM((CB, L), jnp.int32),
            pltpu.VMEM((CB, L), jnp.int32),
            pltpu.VMEM((CB, L), jnp.int32),
            pltpu.VMEM((L, D), jnp.float32),
            pltpu.VMEM((CB, DD), jnp.float32),
            pltpu.SemaphoreType.DMA,
        ],
    )
    def kern(x0_hbm, x1_hbm, x2_hbm, t0_hbm, t1_hbm, t2_hbm, out_hbm,
             idx0_v, idx1_v, idx2_v, rows_v, out_v, sem):
        wid = lax.axis_index("c") * NS + lax.axis_index("s")
        base = wid * ROWS_PER_W

        @pl.loop(0, ROWS_PER_W, step=CB)
        def _chunk(c0):
            row0 = base + c0
            pltpu.sync_copy(x0_hbm.at[pl.ds(row0, CB)], idx0_v)
            pltpu.sync_copy(x1_hbm.at[pl.ds(row0, CB)], idx1_v)
            pltpu.sync_copy(x2_hbm.at[pl.ds(row0, CB)], idx2_v)

            @pl.loop(0, CB)
            def _row(r):
                for t, (tab, idx_v) in enumerate(
                        ((t0_hbm, idx0_v), (t1_hbm, idx1_v), (t2_hbm, idx2_v))):
                    cp0 = pltpu.async_copy(
                        tab.at[idx_v.at[r, pl.ds(0, G0)]],
                        rows_v.at[pl.ds(0, G0)], sem)
                    cp1 = pltpu.async_copy(
                        tab.at[idx_v.at[r, pl.ds(G0, G1)]],
                        rows_v.at[pl.ds(G0, G1)], sem)
                    cp0.wait()
                    cp1.wait()

                    zero = jnp.zeros((LANES,), jnp.float32)

                    def body(i, accs):
                        return tuple(
                            a + rows_v[i, pl.ds(j * LANES, LANES)]
                            for j, a in enumerate(accs))

                    accs = lax.fori_loop(0, L, body, (zero, zero, zero, zero))
                    for j, a in enumerate(accs):
                        out_v[r, pl.ds(t * D + j * LANES, LANES)] = a

            pltpu.sync_copy(out_v, out_hbm.at[pl.ds(row0, CB)])

    return kern(x0, x1, x2, t0, t1, t2)


def _mlp_kernel(x_ref, w1_ref, b1_ref, w2_ref, b2_ref, o_ref):
    x = x_ref[...] * jnp.float32(1.0 / L)
    h = jnp.dot(x, w1_ref[...], preferred_element_type=jnp.float32) + b1_ref[...]
    h = jnp.maximum(h, 0.0)
    o_ref[...] = jnp.dot(h, w2_ref[...], preferred_element_type=jnp.float32) + b2_ref[...]


def _mlp_tc(pooled, W1, b1, W2, b2):
    BLK = 512
    return pl.pallas_call(
        _mlp_kernel,
        grid=(B // BLK,),
        in_specs=[
            pl.BlockSpec((BLK, DD), lambda i: (i, 0)),
            pl.BlockSpec((DD, H), lambda i: (0, 0)),
            pl.BlockSpec((1, H), lambda i: (0, 0)),
            pl.BlockSpec((H, C), lambda i: (0, 0)),
            pl.BlockSpec((1, C), lambda i: (0, 0)),
        ],
        out_specs=pl.BlockSpec((BLK, C), lambda i: (i, 0)),
        out_shape=jax.ShapeDtypeStruct((B, C), jnp.float32),
    )(pooled, W1, b1.reshape(1, H), W2, b2.reshape(1, C))


@jax.jit
def kernel(x0, x1, x2, emb_uni, emb_bi, emb_tri, W1, b1, W2, b2):
    pooled = _pooled_sum_sc(x0, x1, x2, emb_uni, emb_bi, emb_tri)
    return _mlp_tc(pooled, W1, b1, W2, b2)


# speedup vs baseline: 1.2494x; 1.2494x over previous
import functools

import jax
import jax.numpy as jnp
from jax import lax
from jax.experimental import pallas as pl
from jax.experimental.pallas import tpu as pltpu
from jax.experimental.pallas import tpu_sc as plsc

B = 4096
L = 200
D = 64
DD = 3 * D
H = 256
C = 10

NC = 2
NS = 16
NW = NC * NS
ROWS_PER_W = B // NW
CB = 32
LANES = 16
G0 = 128
G1 = L - G0


def _pooled_sum_sc(x0, x1, x2, t0, t1, t2):
    mesh = plsc.VectorSubcoreMesh(core_axis_name="c", subcore_axis_name="s")

    @functools.partial(
        pl.kernel,
        out_type=jax.ShapeDtypeStruct((B, DD), jnp.float32),
        mesh=mesh,
        scratch_types=[
            pltpu.VMEM((CB, L), jnp.int32),
            pltpu.VMEM((CB, L), jnp.int32),
            pltpu.VMEM((CB, L), jnp.int32),
            pltpu.VMEM((L, D), jnp.float32),
            pltpu.VMEM((CB, DD), jnp.float32),
            pltpu.SemaphoreType.DMA,
        ],
        compiler_params=pltpu.CompilerParams(use_tc_tiling_on_sc=False),
    )
    def kern(x0_hbm, x1_hbm, x2_hbm, t0_hbm, t1_hbm, t2_hbm, out_hbm,
             idx0_v, idx1_v, idx2_v, rows_v, out_v, sem):
        wid = lax.axis_index("c") * NS + lax.axis_index("s")
        base = wid * ROWS_PER_W

        @pl.loop(0, ROWS_PER_W, step=CB)
        def _chunk(c0):
            row0 = base + c0
            pltpu.sync_copy(x0_hbm.at[pl.ds(row0, CB)], idx0_v)
            pltpu.sync_copy(x1_hbm.at[pl.ds(row0, CB)], idx1_v)
            pltpu.sync_copy(x2_hbm.at[pl.ds(row0, CB)], idx2_v)

            @pl.loop(0, CB)
            def _row(r):
                for t, (tab, idx_v) in enumerate(
                        ((t0_hbm, idx0_v), (t1_hbm, idx1_v), (t2_hbm, idx2_v))):
                    cp0 = pltpu.async_copy(
                        tab.at[idx_v.at[r, pl.ds(0, G0)]],
                        rows_v.at[pl.ds(0, G0)], sem)
                    cp1 = pltpu.async_copy(
                        tab.at[idx_v.at[r, pl.ds(G0, G1)]],
                        rows_v.at[pl.ds(G0, G1)], sem)
                    cp0.wait()
                    cp1.wait()

                    zero = jnp.zeros((LANES,), jnp.float32)

                    def body(i, accs):
                        return tuple(
                            a + rows_v[i, pl.ds(j * LANES, LANES)]
                            for j, a in enumerate(accs))

                    accs = lax.fori_loop(0, L, body, (zero, zero, zero, zero))
                    for j, a in enumerate(accs):
                        out_v[r, pl.ds(t * D + j * LANES, LANES)] = a

            pltpu.sync_copy(out_v, out_hbm.at[pl.ds(row0, CB)])

    return kern(x0, x1, x2, t0, t1, t2)


def _mlp_kernel(x_ref, w1_ref, b1_ref, w2_ref, b2_ref, o_ref):
    x = x_ref[...] * jnp.float32(1.0 / L)
    h = jnp.dot(x, w1_ref[...], preferred_element_type=jnp.float32) + b1_ref[...]
    h = jnp.maximum(h, 0.0)
    o_ref[...] = jnp.dot(h, w2_ref[...], preferred_element_type=jnp.float32) + b2_ref[...]


def _mlp_tc(pooled, W1, b1, W2, b2):
    BLK = 512
    return pl.pallas_call(
        _mlp_kernel,
        grid=(B // BLK,),
        in_specs=[
            pl.BlockSpec((BLK, DD), lambda i: (i, 0)),
            pl.BlockSpec((DD, H), lambda i: (0, 0)),
            pl.BlockSpec((1, H), lambda i: (0, 0)),
            pl.BlockSpec((H, C), lambda i: (0, 0)),
            pl.BlockSpec((1, C), lambda i: (0, 0)),
        ],
        out_specs=pl.BlockSpec((BLK, C), lambda i: (i, 0)),
        out_shape=jax.ShapeDtypeStruct((B, C), jnp.float32),
    )(pooled, W1, b1.reshape(1, H), W2, b2.reshape(1, C))


@jax.jit
def kernel(x0, x1, x2, emb_uni, emb_bi, emb_tri, W1, b1, W2, b2):
    pooled = _pooled_sum_sc(x0, x1, x2, emb_uni, emb_bi, emb_tri)
    return _mlp_tc(pooled, W1, b1, W2, b2)


# speedup vs baseline: 1.9128x; 1.5310x over previous
import functools

import jax
import jax.numpy as jnp
from jax import lax
from jax.experimental import pallas as pl
from jax.experimental.pallas import tpu as pltpu
from jax.experimental.pallas import tpu_sc as plsc

B = 4096
L = 200
D = 64
DD = 3 * D
H = 256
C = 10
V = 1000000
DP = 128

NC = 2
NS = 16
NW = NC * NS
ROWS_PER_W = B // NW
CB = 32
LANES = 16
G0 = 128
G1 = L - G0

TBLK = 4096


def _detile_kernel(in_ref, o_ref):
    xt = jnp.swapaxes(in_ref[...], 0, 1)
    o_ref[...] = jnp.concatenate(
        [xt, jnp.zeros((xt.shape[0], DP - D), jnp.float32)], axis=1)


def _detile_pad_tc(tab):
    return pl.pallas_call(
        _detile_kernel,
        grid=(pl.cdiv(V, TBLK),),
        in_specs=[pl.BlockSpec((D, TBLK), lambda i: (0, i))],
        out_specs=pl.BlockSpec((TBLK, DP), lambda i: (i, 0)),
        out_shape=jax.ShapeDtypeStruct((V, DP), jnp.float32),
    )(tab.T)


def _pool_one_sc(x, tab):
    mesh = plsc.VectorSubcoreMesh(core_axis_name="c", subcore_axis_name="s")

    @functools.partial(
        pl.kernel,
        out_type=jax.ShapeDtypeStruct((B, D), jnp.float32),
        mesh=mesh,
        scratch_types=[
            pltpu.VMEM((CB, L), jnp.int32),
            pltpu.VMEM((L, DP), jnp.float32),
            pltpu.VMEM((CB, D), jnp.float32),
            pltpu.SemaphoreType.DMA,
        ],
        compiler_params=pltpu.CompilerParams(use_tc_tiling_on_sc=False),
    )
    def kern(x_hbm, t_hbm, out_hbm, idx_v, rows_v, out_v, sem):
        wid = lax.axis_index("c") * NS + lax.axis_index("s")
        base = wid * ROWS_PER_W

        @pl.loop(0, ROWS_PER_W, step=CB)
        def _chunk(c0):
            row0 = base + c0
            pltpu.sync_copy(x_hbm.at[pl.ds(row0, CB)], idx_v)

            @pl.loop(0, CB)
            def _row(r):
                cp0 = pltpu.async_copy(
                    t_hbm.at[idx_v.at[r, pl.ds(0, G0)]],
                    rows_v.at[pl.ds(0, G0)], sem)
                cp1 = pltpu.async_copy(
                    t_hbm.at[idx_v.at[r, pl.ds(G0, G1)]],
                    rows_v.at[pl.ds(G0, G1)], sem)
                cp0.wait()
                cp1.wait()

                zero = jnp.zeros((LANES,), jnp.float32)

                def body(i, accs):
                    return tuple(
                        a + rows_v[i, pl.ds(j * LANES, LANES)]
                        for j, a in enumerate(accs))

                accs = lax.fori_loop(0, L, body, (zero, zero, zero, zero))
                for j, a in enumerate(accs):
                    out_v[r, pl.ds(j * LANES, LANES)] = a

            pltpu.sync_copy(out_v, out_hbm.at[pl.ds(row0, CB)])

    return kern(x, tab)


def _mlp_kernel(x_ref, w1_ref, b1_ref, w2_ref, b2_ref, o_ref):
    x = x_ref[...] * jnp.float32(1.0 / L)
    h = jnp.dot(x, w1_ref[...], preferred_element_type=jnp.float32) + b1_ref[...]
    h = jnp.maximum(h, 0.0)
    o_ref[...] = jnp.dot(h, w2_ref[...], preferred_element_type=jnp.float32) + b2_ref[...]


def _mlp_tc(pooled, W1, b1, W2, b2):
    BLK = 512
    return pl.pallas_call(
        _mlp_kernel,
        grid=(B // BLK,),
        in_specs=[
            pl.BlockSpec((BLK, DD), lambda i: (i, 0)),
            pl.BlockSpec((DD, H), lambda i: (0, 0)),
            pl.BlockSpec((1, H), lambda i: (0, 0)),
            pl.BlockSpec((H, C), lambda i: (0, 0)),
            pl.BlockSpec((1, C), lambda i: (0, 0)),
        ],
        out_specs=pl.BlockSpec((BLK, C), lambda i: (i, 0)),
        out_shape=jax.ShapeDtypeStruct((B, C), jnp.float32),
    )(pooled, W1, b1.reshape(1, H), W2, b2.reshape(1, C))


@jax.jit
def kernel(x0, x1, x2, emb_uni, emb_bi, emb_tri, W1, b1, W2, b2):
    pools = [
        _pool_one_sc(x, _detile_pad_tc(tab))
        for x, tab in ((x0, emb_uni), (x1, emb_bi), (x2, emb_tri))
    ]
    pooled = jnp.concatenate(pools, axis=1)
    return _mlp_tc(pooled, W1, b1, W2, b2)


# speedup vs baseline: 2.2430x; 1.1726x over previous
import functools

import jax
import jax.numpy as jnp
from jax import lax
from jax.experimental import pallas as pl
from jax.experimental.pallas import tpu as pltpu
from jax.experimental.pallas import tpu_sc as plsc

B = 4096
L = 200
D = 64
DD = 3 * D
H = 256
C = 10
V = 1000000
DP = 128

NC = 2
NS = 16
NW = NC * NS
RW = B // NW
LANES = 16
G0 = 128
G1 = L - G0

TBLK = 8192


def _detile_kernel(in_ref, o_ref):
    xt = jnp.swapaxes(in_ref[...], 0, 1)
    o_ref[...] = jnp.concatenate(
        [xt, jnp.zeros((xt.shape[0], DP - D), jnp.float32)], axis=1)


def _detile_pad_tc(tab):
    return pl.pallas_call(
        _detile_kernel,
        grid=(pl.cdiv(V, TBLK),),
        in_specs=[pl.BlockSpec((D, TBLK), lambda i: (0, i))],
        out_specs=pl.BlockSpec((TBLK, DP), lambda i: (i, 0)),
        out_shape=jax.ShapeDtypeStruct((V, DP), jnp.float32),
    )(tab.T)


def _pool_one_sc(x, tab):
    mesh = plsc.VectorSubcoreMesh(core_axis_name="c", subcore_axis_name="s")

    @functools.partial(
        pl.kernel,
        out_type=jax.ShapeDtypeStruct((B, D), jnp.float32),
        mesh=mesh,
        scratch_types=[
            pltpu.VMEM((RW, L), jnp.int32),
            pltpu.VMEM((L, DP), jnp.float32),
            pltpu.VMEM((L, DP), jnp.float32),
            pltpu.VMEM((RW, D), jnp.float32),
            pltpu.SemaphoreType.DMA,
            pltpu.SemaphoreType.DMA,
        ],
        compiler_params=pltpu.CompilerParams(use_tc_tiling_on_sc=False),
    )
    def kern(x_hbm, t_hbm, out_hbm, idx_v, rows_a, rows_b, out_v, sem_a, sem_b):
        wid = lax.axis_index("c") * NS + lax.axis_index("s")
        base = wid * RW

        pltpu.sync_copy(x_hbm.at[pl.ds(base, RW)], idx_v)

        def issue(r, buf, sem):
            pltpu.async_copy(
                t_hbm.at[idx_v.at[r, pl.ds(0, G0)]], buf.at[pl.ds(0, G0)], sem)
            pltpu.async_copy(
                t_hbm.at[idx_v.at[r, pl.ds(G0, G1)]], buf.at[pl.ds(G0, G1)], sem)

        def wait(buf, sem):
            pltpu.make_async_copy(
                t_hbm.at[pl.ds(0, G0)], buf.at[pl.ds(0, G0)], sem).wait()
            pltpu.make_async_copy(
                t_hbm.at[pl.ds(0, G1)], buf.at[pl.ds(G0, G1)], sem).wait()

        def reduce_into(r, buf):
            zero = jnp.zeros((LANES,), jnp.float32)

            def body(i, accs):
                return tuple(
                    a + buf[i, pl.ds(j * LANES, LANES)]
                    for j, a in enumerate(accs))

            accs = lax.fori_loop(0, L, body, (zero, zero, zero, zero),
                                 unroll=4)
            for j, a in enumerate(accs):
                out_v[r, pl.ds(j * LANES, LANES)] = a

        issue(0, rows_a, sem_a)

        @pl.loop(0, RW, step=2)
        def _pair(r):
            issue(r + 1, rows_b, sem_b)
            wait(rows_a, sem_a)
            reduce_into(r, rows_a)
            issue(jnp.minimum(r + 2, RW - 1), rows_a, sem_a)
            wait(rows_b, sem_b)
            reduce_into(r + 1, rows_b)

        wait(rows_a, sem_a)

        pltpu.sync_copy(out_v, out_hbm.at[pl.ds(base, RW)])

    return kern(x, tab)


def _mlp_kernel(x_ref, w1_ref, b1_ref, w2_ref, b2_ref, o_ref):
    x = x_ref[...] * jnp.float32(1.0 / L)
    h = jnp.dot(x, w1_ref[...], preferred_element_type=jnp.float32) + b1_ref[...]
    h = jnp.maximum(h, 0.0)
    o_ref[...] = jnp.dot(h, w2_ref[...], preferred_element_type=jnp.float32) + b2_ref[...]


def _mlp_tc(pooled, W1, b1, W2, b2):
    BLK = 512
    return pl.pallas_call(
        _mlp_kernel,
        grid=(B // BLK,),
        in_specs=[
            pl.BlockSpec((BLK, DD), lambda i: (i, 0)),
            pl.BlockSpec((DD, H), lambda i: (0, 0)),
            pl.BlockSpec((1, H), lambda i: (0, 0)),
            pl.BlockSpec((H, C), lambda i: (0, 0)),
            pl.BlockSpec((1, C), lambda i: (0, 0)),
        ],
        out_specs=pl.BlockSpec((BLK, C), lambda i: (i, 0)),
        out_shape=jax.ShapeDtypeStruct((B, C), jnp.float32),
    )(pooled, W1, b1.reshape(1, H), W2, b2.reshape(1, C))


@jax.jit
def kernel(x0, x1, x2, emb_uni, emb_bi, emb_tri, W1, b1, W2, b2):
    pools = [
        _pool_one_sc(x, _detile_pad_tc(tab))
        for x, tab in ((x0, emb_uni), (x1, emb_bi), (x2, emb_tri))
    ]
    pooled = jnp.concatenate(pools, axis=1)
    return _mlp_tc(pooled, W1, b1, W2, b2)


# speedup vs baseline: 2.9295x; 1.3061x over previous
import functools

import jax
import jax.numpy as jnp
from jax import lax
from jax.experimental import pallas as pl
from jax.experimental.pallas import tpu as pltpu
from jax.experimental.pallas import tpu_sc as plsc

B = 4096
L = 200
D = 64
DD = 3 * D
H = 256
C = 10
V = 1000000
DP = 128

NC = 2
NS = 16
NW = NC * NS
RW = B // NW
LANES = 16
G0 = 128
G1 = L - G0

TBLK = 4096
NB = (V + TBLK - 1) // TBLK
NPAIR = (NB + 1) // 2
VC = NPAIR * TBLK


def _detile_kernel(a_ref, b_ref, o_ref):
    o_ref[...] = jnp.concatenate(
        [jnp.swapaxes(a_ref[...], 0, 1), jnp.swapaxes(b_ref[...], 0, 1)],
        axis=1)


def _detile_pair_tc(tab):
    return pl.pallas_call(
        _detile_kernel,
        grid=(NPAIR,),
        in_specs=[
            pl.BlockSpec((D, TBLK), lambda i: (0, 2 * i)),
            pl.BlockSpec((D, TBLK),
                         lambda i: (0, jnp.minimum(2 * i + 1, NB - 1))),
        ],
        out_specs=pl.BlockSpec((TBLK, DP), lambda i: (i, 0)),
        out_shape=jax.ShapeDtypeStruct((VC, DP), jnp.float32),
    )(tab.T, tab.T)


def _pool_one_sc(h, tab):
    mesh = plsc.VectorSubcoreMesh(core_axis_name="c", subcore_axis_name="s")

    @functools.partial(
        pl.kernel,
        out_type=jax.ShapeDtypeStruct((B, D), jnp.float32),
        mesh=mesh,
        scratch_types=[
            pltpu.VMEM((RW, L), jnp.int32),
            pltpu.VMEM((L, D), jnp.float32),
            pltpu.VMEM((L, D), jnp.float32),
            pltpu.VMEM((RW, D), jnp.float32),
            pltpu.SemaphoreType.DMA,
            pltpu.SemaphoreType.DMA,
        ],
        compiler_params=pltpu.CompilerParams(use_tc_tiling_on_sc=False),
    )
    def kern(h_hbm, t_hbm, out_hbm, idx_v, rows_a, rows_b, out_v, sem_a, sem_b):
        wid = lax.axis_index("c") * NS + lax.axis_index("s")
        base = wid * RW

        pltpu.sync_copy(h_hbm.at[pl.ds(base, RW)], idx_v)

        def issue(r, buf, sem):
            pltpu.async_copy(
                t_hbm.at[idx_v.at[r, pl.ds(0, G0)]], buf.at[pl.ds(0, G0)], sem)
            pltpu.async_copy(
                t_hbm.at[idx_v.at[r, pl.ds(G0, G1)]], buf.at[pl.ds(G0, G1)], sem)

        def wait(buf, sem):
            pltpu.make_async_copy(
                t_hbm.at[pl.ds(0, G0)], buf.at[pl.ds(0, G0)], sem).wait()
            pltpu.make_async_copy(
                t_hbm.at[pl.ds(0, G1)], buf.at[pl.ds(G0, G1)], sem).wait()

        def reduce_into(r, buf):
            zero = jnp.zeros((LANES,), jnp.float32)

            def body(i, accs):
                return tuple(
                    a + buf[i, pl.ds(j * LANES, LANES)]
                    for j, a in enumerate(accs))

            accs = lax.fori_loop(0, L, body, (zero, zero, zero, zero),
                                 unroll=4)
            for j, a in enumerate(accs):
                out_v[r, pl.ds(j * LANES, LANES)] = a

        issue(0, rows_a, sem_a)

        @pl.loop(0, RW, step=2)
        def _pair(r):
            issue(r + 1, rows_b, sem_b)
            wait(rows_a, sem_a)
            reduce_into(r, rows_a)
            issue(jnp.minimum(r + 2, RW - 1), rows_a, sem_a)
            wait(rows_b, sem_b)
            reduce_into(r + 1, rows_b)

        wait(rows_a, sem_a)

        pltpu.sync_copy(out_v, out_hbm.at[pl.ds(base, RW)])

    return kern(h, tab.reshape(2 * VC, D))


def _mlp_kernel(x_ref, w1_ref, b1_ref, w2_ref, b2_ref, o_ref):
    x = x_ref[...] * jnp.float32(1.0 / L)
    h = jnp.dot(x, w1_ref[...], preferred_element_type=jnp.float32) + b1_ref[...]
    h = jnp.maximum(h, 0.0)
    o_ref[...] = jnp.dot(h, w2_ref[...], preferred_element_type=jnp.float32) + b2_ref[...]


def _mlp_tc(pooled, W1, b1, W2, b2):
    BLK = 512
    return pl.pallas_call(
        _mlp_kernel,
        grid=(B // BLK,),
        in_specs=[
            pl.BlockSpec((BLK, DD), lambda i: (i, 0)),
            pl.BlockSpec((DD, H), lambda i: (0, 0)),
            pl.BlockSpec((1, H), lambda i: (0, 0)),
            pl.BlockSpec((H, C), lambda i: (0, 0)),
            pl.BlockSpec((1, C), lambda i: (0, 0)),
        ],
        out_specs=pl.BlockSpec((BLK, C), lambda i: (i, 0)),
        out_shape=jax.ShapeDtypeStruct((B, C), jnp.float32),
    )(pooled, W1, b1.reshape(1, H), W2, b2.reshape(1, C))


@jax.jit
def kernel(x0, x1, x2, emb_uni, emb_bi, emb_tri, W1, b1, W2, b2):
    pools = []
    for x, tab in ((x0, emb_uni), (x1, emb_bi), (x2, emb_tri)):
        q = x >> 12
        h = ((q >> 1) << 13) | ((x & (TBLK - 1)) << 1) | (q & 1)
        pools.append(_pool_one_sc(h, _detile_pair_tc(tab)))
    pooled = jnp.concatenate(pools, axis=1)
    return _mlp_tc(pooled, W1, b1, W2, b2)


# speedup vs baseline: 3.1790x; 1.0852x over previous
import functools

import jax
import jax.numpy as jnp
from jax import lax
from jax.experimental import pallas as pl
from jax.experimental.pallas import tpu as pltpu
from jax.experimental.pallas import tpu_sc as plsc

B = 4096
L = 200
D = 64
DD = 3 * D
H = 256
C = 10
V = 1000000
W = D // 2
DP = 128

NC = 2
NS = 16
NW = NC * NS
RW = B // NW
LANES = 16
G0 = 128
G1 = L - G0

TBLK = 4096
NB = (V + TBLK - 1) // TBLK
NQUAD = (NB + 3) // 4
VC = NQUAD * TBLK


def _pack_words(x):
    lo = x[:W, :].astype(jnp.bfloat16)
    hi = x[W:, :].astype(jnp.bfloat16)
    lou = lax.bitcast_convert_type(lo, jnp.uint16).astype(jnp.uint32)
    hiu = lax.bitcast_convert_type(hi, jnp.uint16).astype(jnp.uint32)
    w = (hiu << 16) | lou
    return lax.bitcast_convert_type(jnp.swapaxes(w, 0, 1), jnp.float32)


def _detile_kernel(a_ref, b_ref, c_ref, d_ref, o_ref):
    o_ref[...] = jnp.concatenate(
        [_pack_words(a_ref[...]), _pack_words(b_ref[...]),
         _pack_words(c_ref[...]), _pack_words(d_ref[...])], axis=1)


def _detile_quad_tc(tab):
    tt = tab.T

    def spec(k):
        return pl.BlockSpec(
            (D, TBLK), lambda i: (0, jnp.minimum(4 * i + k, NB - 1)))

    return pl.pallas_call(
        _detile_kernel,
        grid=(NQUAD,),
        in_specs=[spec(0), spec(1), spec(2), spec(3)],
        out_specs=pl.BlockSpec((TBLK, DP), lambda i: (i, 0)),
        out_shape=jax.ShapeDtypeStruct((VC, DP), jnp.float32),
    )(tt, tt, tt, tt)


def _pool_one_sc(h, tab):
    mesh = plsc.VectorSubcoreMesh(core_axis_name="c", subcore_axis_name="s")

    @functools.partial(
        pl.kernel,
        out_type=jax.ShapeDtypeStruct((B, D), jnp.float32),
        mesh=mesh,
        scratch_types=[
            pltpu.VMEM((RW, L), jnp.int32),
            pltpu.VMEM((L, W), jnp.float32),
            pltpu.VMEM((L, W), jnp.float32),
            pltpu.VMEM((RW, D), jnp.float32),
            pltpu.SemaphoreType.DMA,
            pltpu.SemaphoreType.DMA,
        ],
        compiler_params=pltpu.CompilerParams(
            use_tc_tiling_on_sc=False, needs_layout_passes=False),
    )
    def kern(h_hbm, t_hbm, out_hbm, idx_v, rows_a, rows_b, out_v, sem_a, sem_b):
        wid = lax.axis_index("c") * NS + lax.axis_index("s")
        base = wid * RW

        pltpu.sync_copy(h_hbm.at[pl.ds(base, RW)], idx_v)

        def issue(r, buf, sem):
            pltpu.async_copy(
                t_hbm.at[idx_v.at[r, pl.ds(0, G0)]], buf.at[pl.ds(0, G0)], sem)
            pltpu.async_copy(
                t_hbm.at[idx_v.at[r, pl.ds(G0, G1)]], buf.at[pl.ds(G0, G1)], sem)

        def wait(buf, sem):
            pltpu.make_async_copy(
                t_hbm.at[pl.ds(0, G0)], buf.at[pl.ds(0, G0)], sem).wait()
            pltpu.make_async_copy(
                t_hbm.at[pl.ds(0, G1)], buf.at[pl.ds(G0, G1)], sem).wait()

        def reduce_into(r, buf):
            zero = jnp.zeros((LANES,), jnp.float32)

            def body(i, accs):
                a0, a1, a2, a3 = accs
                b0 = plsc.bitcast(buf[i, pl.ds(0, LANES)], jnp.bfloat16)
                b1 = plsc.bitcast(buf[i, pl.ds(LANES, LANES)], jnp.bfloat16)
                lo0, hi0 = plsc.unpack(
                    b0, format=plsc.PackFormat.INTERLEAVED,
                    preferred_element_type=jnp.float32)
                lo1, hi1 = plsc.unpack(
                    b1, format=plsc.PackFormat.INTERLEAVED,
                    preferred_element_type=jnp.float32)
                return (a0 + lo0, a1 + lo1, a2 + hi0, a3 + hi1)

            accs = lax.fori_loop(0, L, body, (zero, zero, zero, zero),
                                 unroll=4)
            for j, a in enumerate(accs):
                out_v[r, pl.ds(j * LANES, LANES)] = a

        issue(0, rows_a, sem_a)

        @pl.loop(0, RW, step=2)
        def _pair(r):
            issue(r + 1, rows_b, sem_b)
            wait(rows_a, sem_a)
            reduce_into(r, rows_a)
            issue(jnp.minimum(r + 2, RW - 1), rows_a, sem_a)
            wait(rows_b, sem_b)
            reduce_into(r + 1, rows_b)

        wait(rows_a, sem_a)

        pltpu.sync_copy(out_v, out_hbm.at[pl.ds(base, RW)])

    return kern(h, tab.reshape(4 * VC, W))


def _mlp_kernel(x_ref, w1_ref, b1_ref, w2_ref, b2_ref, o_ref):
    x = x_ref[...] * jnp.float32(1.0 / L)
    h = jnp.dot(x, w1_ref[...], preferred_element_type=jnp.float32) + b1_ref[...]
    h = jnp.maximum(h, 0.0)
    o_ref[...] = jnp.dot(h, w2_ref[...], preferred_element_type=jnp.float32) + b2_ref[...]


def _mlp_tc(pooled, W1, b1, W2, b2):
    BLK = 512
    return pl.pallas_call(
        _mlp_kernel,
        grid=(B // BLK,),
        in_specs=[
            pl.BlockSpec((BLK, DD), lambda i: (i, 0)),
            pl.BlockSpec((DD, H), lambda i: (0, 0)),
            pl.BlockSpec((1, H), lambda i: (0, 0)),
            pl.BlockSpec((H, C), lambda i: (0, 0)),
            pl.BlockSpec((1, C), lambda i: (0, 0)),
        ],
        out_specs=pl.BlockSpec((BLK, C), lambda i: (i, 0)),
        out_shape=jax.ShapeDtypeStruct((B, C), jnp.float32),
    )(pooled, W1, b1.reshape(1, H), W2, b2.reshape(1, C))


@jax.jit
def kernel(x0, x1, x2, emb_uni, emb_bi, emb_tri, W1, b1, W2, b2):
    pools = []
    for x, tab in ((x0, emb_uni), (x1, emb_bi), (x2, emb_tri)):
        q = x >> 12
        h = ((q >> 2) << 14) | ((x & (TBLK - 1)) << 2) | (q & 3)
        pools.append(_pool_one_sc(h, _detile_quad_tc(tab)))
    pooled = jnp.concatenate(pools, axis=1)
    return _mlp_tc(pooled, W1, b1, W2, b2)
